# baseline (speedup 1.0000x reference)
import functools

import jax
import jax.numpy as jnp
from jax import lax
from jax.experimental import pallas as pl
from jax.experimental.pallas import tpu as pltpu
from jax.experimental.pallas import tpu_sc as plsc

_N = 32768
_B = 128
_L = 16
_CHUNKS = _N // _L
_NW = 32
_ROWS_PER_W = _B // _NW
_EPS = 1e-8
_INT_MIN = -2147483648
_TB = 12


def _key_to_f32(k):
    i = jnp.where(k < 0, k ^ jnp.int32(0x7FFFFFFF), k)
    return lax.bitcast_convert_type(i, jnp.float32)


def _sc_body(x_hbm, r_hbm, o_hbm, xv, kv, nv, rv):
    c = lax.axis_index("c")
    s = lax.axis_index("s")
    wid = s * 2 + c
    pltpu.sync_copy(r_hbm, rv)

    for ri in range(_ROWS_PER_W):
        row = wid * _ROWS_PER_W + ri
        pltpu.sync_copy(x_hbm.at[row], xv)
        rchunk = rv[pl.ds((row // _L) * _L, _L)]
        lane = lax.iota(jnp.int32, _L)
        rrow = jnp.broadcast_to(
            jnp.max(
                jnp.where(lane == row % _L, rchunk, jnp.float32(-jnp.inf))
            ),
            (_L,),
        )

        def mx_body(i, acc):
            return jnp.maximum(acc, xv[pl.ds(i * _L, _L)])

        mvec = lax.fori_loop(
            0, _CHUNKS, mx_body, jnp.full((_L,), -jnp.inf, jnp.float32)
        )
        m = jnp.broadcast_to(jnp.max(mvec), (_L,))

        def key_body(i, carry):
            zc, nm = carry
            v = xv[pl.ds(i * _L, _L)]
            im = v - m
            zm = jnp.exp(im) == 0.0
            zc = zc + plsc.all_reduce_population_count(zm)
            nm = nm + plsc.all_reduce_population_count(im == 0.0)
            xmm = im * (1.0 - zm.astype(jnp.float32))
            ib = lax.bitcast_convert_type(xmm, jnp.int32)
            kv[pl.ds(i * _L, _L)] = jnp.where(
                ib < 0, ib ^ jnp.int32(0x7FFFFFFF), ib
            )
            return zc, nm

        zc, nm = lax.fori_loop(
            0,
            _CHUNKS,
            key_body,
            (jnp.zeros((_L,), jnp.int32), jnp.zeros((_L,), jnp.int32)),
        )

        zf = zc.astype(jnp.float32) * (1.0 / _N)
        q = jnp.clip((rrow - zf) / (1.0 - zf), 0.0, 1.0)
        idx = q * jnp.float32(_N - 1)
        k = idx.astype(jnp.int32)
        frac = idx - k.astype(jnp.float32)
        ps0 = jnp.where(
            jnp.full((_L,), _N, jnp.int32) - nm <= k,
            jnp.zeros((_L,), jnp.int32),
            jnp.full((_L,), jnp.int32(_INT_MIN)),
        )

        def search(it, ps):
            cand = ps + (jnp.int32(1) << (jnp.int32(30) - it))

            def cnt_body(i, a):
                kk = kv[pl.ds(i * _L, _L)]
                return a + plsc.all_reduce_population_count(kk < cand)

            cnt = lax.fori_loop(
                0, _CHUNKS, cnt_body, jnp.zeros((_L,), jnp.int32)
            )
            return jnp.where(cnt <= k, cand, ps)

        ps = lax.fori_loop(0, 31 - _TB, search, ps0)

        def ah_body(i, carry):
            cl, gm = carry
            kk = kv[pl.ds(i * _L, _L)]
            ltn = kk < ps + jnp.int32(1 << _TB)
            cl = cl + plsc.all_reduce_population_count(ltn)
            gm = jnp.minimum(gm, jnp.where(ltn, jnp.int32(2147483647), kk))
            return cl, gm

        cl, gmv = lax.fori_loop(
            0,
            _CHUNKS,
            ah_body,
            (
                jnp.zeros((_L,), jnp.int32),
                jnp.full((_L,), jnp.int32(2147483647)),
            ),
        )
        gm = jnp.broadcast_to(jnp.min(gmv), (_L,)) & jnp.int32(
            ~((1 << _TB) - 1)
        )
        a_low = _key_to_f32(ps)
        a_high = _key_to_f32(jnp.where(cl >= k + 2, ps, gm))
        a_high = jnp.where(frac > 0.0, a_high, a_low)
        quant = a_low * (1.0 - frac) + a_high * frac
        t = _EPS - quant

        def sm1(i, dn):
            v = xv[pl.ds(i * _L, _L)]
            im = v - m
            num = jnp.exp(im) * jnp.maximum(im + t, 0.0)
            nv[pl.ds(i * _L, _L)] = num
            return dn + num

        dnv = lax.fori_loop(0, _CHUNKS, sm1, jnp.zeros((_L,), jnp.float32))
        inv = jnp.full((_L,), 1.0, jnp.float32) / jnp.broadcast_to(
            jnp.sum(dnv), (_L,)
        )

        def sm2(i, z):
            nv[pl.ds(i * _L, _L)] = nv[pl.ds(i * _L, _L)] * inv
            return z

        lax.fori_loop(0, _CHUNKS, sm2, jnp.int32(0))
        pltpu.sync_copy(nv, o_hbm.at[row])


@jax.jit
def _rsoftmax_sc(x, r):
    mesh = plsc.VectorSubcoreMesh(core_axis_name="c", subcore_axis_name="s")
    kfn = functools.partial(
        pl.kernel,
        mesh=mesh,
        out_type=jax.ShapeDtypeStruct((_B, _N), jnp.float32),
        scratch_types=[
            pltpu.VMEM((_N,), jnp.float32),
            pltpu.VMEM((_N,), jnp.int32),
            pltpu.VMEM((_N,), jnp.float32),
            pltpu.VMEM((_B,), jnp.float32),
        ],
        compiler_params=pltpu.CompilerParams(needs_layout_passes=False),
    )(_sc_body)
    return kfn(x, r.reshape(_B))


def kernel(input, r):
    return _rsoftmax_sc(input, r)


# speedup vs baseline: 12.8667x; 12.8667x over previous
import jax
import jax.numpy as jnp
from jax.experimental import pallas as pl
from jax.experimental.pallas import tpu as pltpu

_N = 32768
_ROWS_PER_BLOCK = 64
_EPS = 1e-8
_INT_MIN = -2147483648
_TB = 14


def _f32_to_key(x):
    i = jax.lax.bitcast_convert_type(x, jnp.int32)
    return jnp.where(i < 0, i ^ jnp.int32(0x7FFFFFFF), i)


def _key_to_f32(k):
    i = jnp.where(k < 0, k ^ jnp.int32(0x7FFFFFFF), k)
    return jax.lax.bitcast_convert_type(i, jnp.float32)


def _rsoftmax_block(x_ref, r_ref, o_ref, key_ref):
    x = x_ref[...]
    m = jnp.max(x, axis=1, keepdims=True)
    im = x - m
    zmask = jnp.exp(im) == 0.0
    zcnt = jnp.sum(zmask, axis=1, keepdims=True)
    nmax = jnp.sum(im == 0.0, axis=1, keepdims=True)
    zf = zcnt.astype(jnp.float32) * (1.0 / _N)
    r = r_ref[...]
    q = jnp.clip((r - zf) / (1.0 - zf), 0.0, 1.0)
    idx = q * jnp.float32(_N - 1)
    kf = jnp.floor(idx)
    frac = idx - kf
    k = kf.astype(jnp.int32)

    key_ref[...] = _f32_to_key(im * (1.0 - zmask.astype(jnp.float32)))

    ps0 = jnp.where(
        _N - nmax <= k,
        jnp.zeros_like(k),
        jnp.full_like(k, jnp.int32(_INT_MIN)),
    )

    def step(i, ps):
        cand = ps + (jnp.int32(1) << (jnp.int32(30) - i))
        cnt = jnp.sum(key_ref[...] < cand, axis=1, keepdims=True)
        return jnp.where(cnt <= k, cand, ps)

    ps = jax.lax.fori_loop(0, 31 - _TB, step, ps0)
    half = jnp.int32(1 << (_TB - 1))
    a_low = _key_to_f32(ps + half)

    keys = key_ref[...]
    lt_next = keys < ps + jnp.int32(1 << _TB)
    cnt_le = jnp.sum(lt_next, axis=1, keepdims=True)
    gmin = jnp.min(
        jnp.where(lt_next, jnp.int32(2147483647), keys),
        axis=1,
        keepdims=True,
    ) & jnp.int32(~((1 << _TB) - 1))
    a_high = _key_to_f32(jnp.where(cnt_le >= k + 2, ps, gmin) + half)
    a_high = jnp.where(frac > 0.0, a_high, a_low)

    quant = a_low * (1.0 - frac) + a_high * frac
    t = _EPS - quant

    im2 = x_ref[...] - m
    num = jnp.exp(im2) * jnp.maximum(im2 + t, 0.0)
    o_ref[...] = num
    denom = jnp.sum(num, axis=1, keepdims=True)
    o_ref[...] = o_ref[...] * (1.0 / denom)


@jax.jit
def _rsoftmax(x, r):
    grid = (x.shape[0] // _ROWS_PER_BLOCK,)
    return pl.pallas_call(
        _rsoftmax_block,
        grid=grid,
        in_specs=[
            pl.BlockSpec((_ROWS_PER_BLOCK, _N), lambda i: (i, 0)),
            pl.BlockSpec((_ROWS_PER_BLOCK, 1), lambda i: (i, 0)),
        ],
        out_specs=pl.BlockSpec((_ROWS_PER_BLOCK, _N), lambda i: (i, 0)),
        out_shape=jax.ShapeDtypeStruct(x.shape, jnp.float32),
        scratch_shapes=[
            pltpu.VMEM((_ROWS_PER_BLOCK, _N), jnp.int32),
        ],
        compiler_params=pltpu.CompilerParams(
            dimension_semantics=("parallel",),
        ),
    )(x, r)


def kernel(input, r):
    return _rsoftmax(input, r)


# speedup vs baseline: 13.3503x; 1.0376x over previous
import jax
import jax.numpy as jnp
from jax.experimental import pallas as pl
from jax.experimental.pallas import tpu as pltpu

_N = 32768
_ROWS_PER_BLOCK = 64
_EPS = 1e-8
_INT_MIN = -2147483648
_TB = 14


def _f32_to_key(x):
    i = jax.lax.bitcast_convert_type(x, jnp.int32)
    return jnp.where(i < 0, i ^ jnp.int32(0x7FFFFFFF), i)


def _key_to_f32(k):
    i = jnp.where(k < 0, k ^ jnp.int32(0x7FFFFFFF), k)
    return jax.lax.bitcast_convert_type(i, jnp.float32)


def _rsoftmax_block(x_ref, r_ref, o_ref, key_ref):
    x = x_ref[...]
    m = jnp.max(x, axis=1, keepdims=True)
    im = x - m
    zmask = jnp.exp(im) == 0.0
    zcnt = jnp.sum(zmask, axis=1, keepdims=True)
    nmax = jnp.sum(im == 0.0, axis=1, keepdims=True)
    zf = zcnt.astype(jnp.float32) * (1.0 / _N)
    r = r_ref[...]
    q = jnp.clip((r - zf) / (1.0 - zf), 0.0, 1.0)
    idx = q * jnp.float32(_N - 1)
    kf = jnp.floor(idx)
    frac = idx - kf
    k = kf.astype(jnp.int32)

    key_ref[...] = _f32_to_key(im * (1.0 - zmask.astype(jnp.float32)))

    ps0 = jnp.where(
        _N - nmax <= k,
        jnp.zeros_like(k),
        jnp.full_like(k, jnp.int32(_INT_MIN)),
    )

    ps = ps0
    for bit in range(30, _TB - 1, -1):
        cand = ps + jnp.int32(1 << bit)
        cnt = jnp.sum(key_ref[...] < cand, axis=1, keepdims=True)
        ps = jnp.where(cnt <= k, cand, ps)
    half = jnp.int32(1 << (_TB - 1))
    a_low = _key_to_f32(ps + half)

    keys = key_ref[...]
    lt_next = keys < ps + jnp.int32(1 << _TB)
    cnt_le = jnp.sum(lt_next, axis=1, keepdims=True)
    gmin = jnp.min(
        jnp.where(lt_next, jnp.int32(2147483647), keys),
        axis=1,
        keepdims=True,
    ) & jnp.int32(~((1 << _TB) - 1))
    a_high = _key_to_f32(jnp.where(cnt_le >= k + 2, ps, gmin) + half)
    a_high = jnp.where(frac > 0.0, a_high, a_low)

    quant = a_low * (1.0 - frac) + a_high * frac
    t = _EPS - quant

    im2 = x_ref[...] - m
    num = jnp.exp(im2) * jnp.maximum(im2 + t, 0.0)
    o_ref[...] = num
    denom = jnp.sum(num, axis=1, keepdims=True)
    o_ref[...] = o_ref[...] * (1.0 / denom)


@jax.jit
def _rsoftmax(x, r):
    grid = (x.shape[0] // _ROWS_PER_BLOCK,)
    return pl.pallas_call(
        _rsoftmax_block,
        grid=grid,
        in_specs=[
            pl.BlockSpec((_ROWS_PER_BLOCK, _N), lambda i: (i, 0)),
            pl.BlockSpec((_ROWS_PER_BLOCK, 1), lambda i: (i, 0)),
        ],
        out_specs=pl.BlockSpec((_ROWS_PER_BLOCK, _N), lambda i: (i, 0)),
        out_shape=jax.ShapeDtypeStruct(x.shape, jnp.float32),
        scratch_shapes=[
            pltpu.VMEM((_ROWS_PER_BLOCK, _N), jnp.int32),
        ],
        compiler_params=pltpu.CompilerParams(
            dimension_semantics=("parallel",),
        ),
    )(x, r)


def kernel(input, r):
    return _rsoftmax(input, r)
